# baseline (speedup 1.0000x reference)
import functools

import jax
import jax.numpy as jnp
import numpy as np
from jax.experimental import pallas as pl
from jax.experimental.pallas import tpu as pltpu

ROWS = 128
COLS = 100000
BR = 8
BC = 8192
NCB = (COLS + BC - 1) // BC

_U32 = jnp.uint32
_TINY = np.float32(np.finfo(np.float32).tiny)
_NEG_INF = np.float32(-np.inf)


def _threefry_xor_fold(x1):
    k0 = np.uint32(0)
    k1 = np.uint32(42)
    ks = (k0, k1, np.uint32(k0 ^ k1 ^ np.uint32(0x1BD11BDA)))
    rot = ((13, 15, 26, 6), (17, 29, 16, 24))

    x0 = jnp.zeros_like(x1) + ks[0]
    x1 = x1 + ks[1]
    for n in range(5):
        for r in rot[n % 2]:
            x0 = x0 + x1
            x1 = (x1 << _U32(r)) | (x1 >> _U32(32 - r))
            x1 = x1 ^ x0
        x0 = x0 + ks[(n + 1) % 3]
        x1 = x1 + ks[(n + 2) % 3] + _U32(n + 1)
    return x0 ^ x1


def _gumbel_from_bits(bits):
    fl = jax.lax.bitcast_convert_type(
        (bits >> _U32(9)) | _U32(0x3F800000), jnp.float32
    )
    u = fl - np.float32(1.0)
    u = jnp.maximum(_TINY, u * (np.float32(1.0) - _TINY) + _TINY)
    return -jnp.log(-jnp.log(u))


def _body(x_ref, out_ref, bestv, besti):
    c = pl.program_id(1)
    r = pl.program_id(0)

    @pl.when(c == 0)
    def _init():
        bestv[...] = jnp.full_like(bestv, _NEG_INF)
        besti[...] = jnp.zeros_like(besti)

    x = x_ref[...]

    row = (r * BR + jax.lax.broadcasted_iota(jnp.int32, (BR, BC), 0)).astype(_U32)
    col = (c * BC + jax.lax.broadcasted_iota(jnp.int32, (BR, BC), 1)).astype(_U32)
    flat = row * _U32(COLS) + col

    g = _gumbel_from_bits(_threefry_xor_fold(flat))
    val = x + g
    valid = col < _U32(COLS)
    val = jnp.where(valid, val, _NEG_INF)

    m = jnp.max(val, axis=1, keepdims=True)
    cand = jnp.where(val == m, col.astype(jnp.int32), jnp.int32(COLS))
    a = jnp.min(cand, axis=1, keepdims=True)

    old_v = bestv[:, :1]
    old_i = besti[:, :1]
    upd = m > old_v
    bestv[:, :1] = jnp.where(upd, m, old_v)
    besti[:, :1] = jnp.where(upd, a, old_i)

    @pl.when(c == NCB - 1)
    def _emit():
        out_ref[...] = besti[:, :1]


@jax.jit
def _run(outputs):
    out = pl.pallas_call(
        _body,
        grid=(ROWS // BR, NCB),
        in_specs=[pl.BlockSpec((BR, BC), lambda r, c: (r, c))],
        out_specs=pl.BlockSpec((BR, 1), lambda r, c: (r, 0)),
        out_shape=jax.ShapeDtypeStruct((ROWS, 1), jnp.int32),
        scratch_shapes=[
            pltpu.VMEM((BR, 128), jnp.float32),
            pltpu.VMEM((BR, 128), jnp.int32),
        ],
        compiler_params=pltpu.CompilerParams(
            dimension_semantics=("parallel", "arbitrary"),
        ),
    )(outputs)
    return out[:, 0]


def kernel(outputs):
    return _run(outputs)
